# baseline (speedup 1.0000x reference)
import dataclasses
import functools

import jax
import jax.numpy as jnp
from jax import lax
from jax.experimental import pallas as pl
from jax.experimental.pallas import tpu as pltpu
from jax.experimental.pallas import tpu_sc as plsc

NC = 2
NS = 16
NW = NC * NS
LANES = 16
CHUNK = 128
ZROWS = 40
WROWS = 1000


def _sc_aggregate(edge_index, x, n, e, d):
    nch = e // CHUNK
    nfull = nch // NW
    nextra = nch - nfull * NW
    nzb = n // ZROWS
    zrounds = (nzb + NS - 1) // NS

    mesh = plsc.VectorSubcoreMesh(core_axis_name="c", subcore_axis_name="s")

    @functools.partial(
        pl.kernel,
        out_type=(
            jax.ShapeDtypeStruct((NC, n, d), jnp.float32),
            jax.ShapeDtypeStruct((NW * n,), jnp.float32),
        ),
        mesh=mesh,
        scratch_types=[
            pltpu.VMEM_SHARED((n, d), jnp.float32),
            pltpu.VMEM((4, 2, CHUNK), jnp.int32),
            pltpu.VMEM((2, CHUNK, d), jnp.float32),
            pltpu.VMEM((n,), jnp.float32),
            pltpu.VMEM((ZROWS, d), jnp.float32),
            pltpu.SemaphoreType.DMA,
            pltpu.SemaphoreType.DMA,
            pltpu.SemaphoreType.DMA,
            pltpu.SemaphoreType.DMA,
            pltpu.SemaphoreType.DMA,
            pltpu.SemaphoreType.DMA,
        ],
        compiler_params=dataclasses.replace(
            pltpu.CompilerParams(), needs_layout_passes=False),
    )
    def sc_kernel(edge_hbm, x_hbm, agg_hbm, cnt_hbm,
                  agg_sh, ev, rows, cnt_v, z128_v, si0, si1, sg0, sg1,
                  ss0, ss1):
        cid = lax.axis_index("c")
        sid = lax.axis_index("s")
        wid = cid * NS + sid
        sem_i = (si0, si1)
        sem_g = (sg0, sg1)
        sem_s = (ss0, ss1)

        @pl.loop(0, ZROWS)
        def _(r):
            @pl.loop(0, d, step=LANES)
            def _(cc):
                z128_v[r, pl.ds(cc, LANES)] = jnp.zeros((LANES,), jnp.float32)

        def zero_copy(bj):
            return pltpu.make_async_copy(
                z128_v, agg_sh.at[pl.ds(bj * ZROWS, ZROWS)], si0)

        @pl.loop(0, zrounds)
        def _(k):
            bj = sid + NS * k

            @pl.when(bj < nzb)
            def _():
                zero_copy(bj).start()

        @pl.loop(0, n, step=LANES)
        def _(i):
            cnt_v[pl.ds(i, LANES)] = jnp.zeros((LANES,), jnp.float32)

        @pl.loop(0, zrounds)
        def _(k):
            bj = sid + NS * k

            @pl.when(bj < nzb)
            def _():
                zero_copy(bj).wait()

        plsc.subcore_barrier()

        base = wid * nfull * CHUNK
        ones16 = jnp.ones((LANES,), jnp.float32)

        def count_dst(dst_ref):
            nlanes = dst_ref.shape[0]

            @pl.loop(0, nlanes // LANES)
            def _(j):
                idx16 = dst_ref[pl.ds(j * LANES, LANES)]
                plsc.addupdate_scatter(cnt_v, [idx16], ones16)

        def idx_copy(ci, s):
            return pltpu.make_async_copy(
                edge_hbm.at[:, pl.ds(base + ci * CHUNK, CHUNK)],
                ev.at[s], sem_i[s % 2])

        def gather_copy(b, s):
            return pltpu.make_async_copy(
                x_hbm.at[ev.at[s, 0]], rows.at[b], sem_g[b])

        def scat_copy(b, s):
            return pltpu.make_async_copy(
                rows.at[b], agg_sh.at[ev.at[s, 1]], sem_s[b])

        def scat_start(b, s):
            pltpu.async_copy(rows.at[b], agg_sh.at[ev.at[s, 1]], sem_s[b],
                             add=True)

        idx_copy(0, 0).start()
        idx_copy(1, 1).start()

        @pl.loop(0, nfull - 2, step=4)
        def _(i):
            for off in range(4):
                ci = i + off
                b = off % 2
                s = off
                pb = 1 - b
                ps = (off + 3) % 4
                idx_copy(ci, s).wait()

                @pl.when(ci >= 2)
                def _():
                    scat_copy(b, (off + 2) % 4).wait()

                gather_copy(b, s).start()

                @pl.when(ci > 0)
                def _():
                    gather_copy(pb, ps).wait()
                    scat_start(pb, ps)
                    count_dst(ev.at[ps, 1])

                idx_copy(ci + 2, (off + 2) % 4).start()

        for ci in (nfull - 2, nfull - 1):
            b, s = ci % 2, ci % 4
            pb, ps = 1 - ci % 2, (ci + 3) % 4
            idx_copy(ci, s).wait()
            scat_copy(b, (ci + 2) % 4).wait()
            gather_copy(b, s).start()
            gather_copy(pb, ps).wait()
            scat_start(pb, ps)
            count_dst(ev.at[ps, 1])
        lb, ls = (nfull - 1) % 2, (nfull - 1) % 4
        gather_copy(lb, ls).wait()
        scat_start(lb, ls)
        count_dst(ev.at[ls, 1])
        scat_copy(1 - lb, (ls + 3) % 4).wait()
        scat_copy(lb, ls).wait()

        if nextra:
            @pl.when(wid < nextra)
            def _():
                xoff = (nfull * NW + wid) * CHUNK
                pltpu.sync_copy(edge_hbm.at[:, pl.ds(xoff, CHUNK)], ev.at[0])
                pltpu.async_copy(x_hbm.at[ev.at[0, 0]], rows.at[0], sg0).wait()
                pltpu.sync_copy(rows.at[0], agg_sh.at[ev.at[0, 1]], add=True)
                count_dst(ev.at[0, 1])

        plsc.subcore_barrier()

        nwb = n // WROWS

        def wout_copy():
            rsl = pl.ds(sid * WROWS, WROWS)
            return pltpu.make_async_copy(
                agg_sh.at[rsl], agg_hbm.at[cid].at[rsl], sg0)

        @pl.when(sid < nwb)
        def _():
            wout_copy().start()

        pltpu.async_copy(cnt_v, cnt_hbm.at[pl.ds(wid * n, n)], sg1)
        pltpu.make_async_copy(cnt_v, cnt_hbm.at[pl.ds(wid * n, n)],
                              sg1).wait()

        @pl.when(sid < nwb)
        def _():
            wout_copy().wait()

    return sc_kernel(edge_index, x)


def _tc_head(agg2, cnt2, x, W_l, b_l, W_r, W_score, b_score, rs, alpha):
    n, d = x.shape

    blk = 2000
    ngrid = n // blk

    def recip_body(cnt_ref, out_ref):
        cnt = jnp.sum(cnt_ref[...], axis=0)
        out_ref[...] = (1.0 / jnp.maximum(cnt, 1.0)).reshape(n, 1)

    recip = pl.pallas_call(
        recip_body,
        out_shape=jax.ShapeDtypeStruct((n, 1), jnp.float32),
    )(cnt2.reshape(NW, n))

    def body(agg_ref, recip_ref, x_ref, wl_ref, bl_ref, wr_ref, ws_ref,
             bs_ref, rs_ref, alpha_ref, out_ref):
        agg = agg_ref[0] + agg_ref[1]
        mean = agg * recip_ref[...]
        h = lax.dot_general(mean, wl_ref[...], (((1,), (1,)), ((), ())),
                            preferred_element_type=jnp.float32)
        h = h + lax.dot_general(x_ref[...], wr_ref[...],
                                (((1,), (1,)), ((), ())),
                                preferred_element_type=jnp.float32)
        h = jnp.maximum(h + bl_ref[...], 0.0)
        g = lax.dot_general(h, ws_ref[...], (((1,), (0,)), ((), ())),
                            preferred_element_type=jnp.float32)
        g = g + bs_ref[0, 0]
        a = jax.nn.sigmoid(alpha_ref[0, 0])
        out_ref[...] = a * rs_ref[...] + (1.0 - a) * g

    return pl.pallas_call(
        body,
        grid=(ngrid,),
        in_specs=[
            pl.BlockSpec((NC, blk, d), lambda i: (0, i, 0)),
            pl.BlockSpec((blk, 1), lambda i: (i, 0)),
            pl.BlockSpec((blk, d), lambda i: (i, 0)),
            pl.BlockSpec((d, d), lambda i: (0, 0)),
            pl.BlockSpec((1, d), lambda i: (0, 0)),
            pl.BlockSpec((d, d), lambda i: (0, 0)),
            pl.BlockSpec((d, 1), lambda i: (0, 0)),
            pl.BlockSpec((1, 1), lambda i: (0, 0)),
            pl.BlockSpec((blk, 1), lambda i: (i, 0)),
            pl.BlockSpec((1, 1), lambda i: (0, 0)),
        ],
        out_specs=pl.BlockSpec((blk, 1), lambda i: (i, 0)),
        out_shape=jax.ShapeDtypeStruct((n, 1), jnp.float32),
    )(agg2, recip, x, W_l, b_l.reshape(1, -1), W_r,
      W_score.reshape(-1, 1), b_score.reshape(1, 1), rs.reshape(n, 1),
      alpha.reshape(1, 1))


def kernel(x, edge_index, reranker_scores, W_l, b_l, W_r, W_score, b_score,
           alpha):
    n, d = x.shape
    e = edge_index.shape[1]
    agg2, cnt2 = _sc_aggregate(edge_index, x, n, e, d)
    out = _tc_head(agg2, cnt2, x, W_l, b_l, W_r, W_score, b_score,
                   reranker_scores, alpha)
    return out[:, 0]


# speedup vs baseline: 1.0389x; 1.0389x over previous
import dataclasses
import functools

import jax
import jax.numpy as jnp
from jax import lax
from jax.experimental import pallas as pl
from jax.experimental.pallas import tpu as pltpu
from jax.experimental.pallas import tpu_sc as plsc

NC = 2
NS = 16
NW = NC * NS
LANES = 16
CHUNK = 128
ZROWS = 40
WROWS = 1000


def _sc_aggregate(edge_index, x, n, e, d):
    nch = e // CHUNK
    nfull = nch // NW
    nextra = nch - nfull * NW
    nzb = n // ZROWS
    zrounds = (nzb + NS - 1) // NS

    mesh = plsc.VectorSubcoreMesh(core_axis_name="c", subcore_axis_name="s")

    @functools.partial(
        pl.kernel,
        out_type=(
            jax.ShapeDtypeStruct((NC, n, d), jnp.float32),
            jax.ShapeDtypeStruct((NW * n,), jnp.float32),
        ),
        mesh=mesh,
        scratch_types=[
            pltpu.VMEM_SHARED((n, d), jnp.float32),
            pltpu.VMEM((4, 2, CHUNK), jnp.int32),
            pltpu.VMEM((2, CHUNK, d), jnp.float32),
            pltpu.VMEM((n,), jnp.float32),
            pltpu.VMEM((ZROWS, d), jnp.float32),
            pltpu.SemaphoreType.DMA,
            pltpu.SemaphoreType.DMA,
            pltpu.SemaphoreType.DMA,
            pltpu.SemaphoreType.DMA,
            pltpu.SemaphoreType.DMA,
            pltpu.SemaphoreType.DMA,
        ],
        compiler_params=dataclasses.replace(
            pltpu.CompilerParams(), needs_layout_passes=False),
    )
    def sc_kernel(edge_hbm, x_hbm, agg_hbm, cnt_hbm,
                  agg_sh, ev, rows, cnt_v, z128_v, si0, si1, sg0, sg1,
                  ss0, ss1):
        cid = lax.axis_index("c")
        sid = lax.axis_index("s")
        wid = cid * NS + sid
        sem_i = (si0, si1)
        sem_g = (sg0, sg1)
        sem_s = (ss0, ss1)

        @pl.loop(0, ZROWS)
        def _(r):
            @pl.loop(0, d, step=LANES)
            def _(cc):
                z128_v[r, pl.ds(cc, LANES)] = jnp.zeros((LANES,), jnp.float32)

        def zero_copy(bj):
            return pltpu.make_async_copy(
                z128_v, agg_sh.at[pl.ds(bj * ZROWS, ZROWS)], si0)

        @pl.loop(0, zrounds)
        def _(k):
            bj = sid + NS * k

            @pl.when(bj < nzb)
            def _():
                zero_copy(bj).start()

        @pl.loop(0, n, step=LANES)
        def _(i):
            cnt_v[pl.ds(i, LANES)] = jnp.zeros((LANES,), jnp.float32)

        @pl.loop(0, zrounds)
        def _(k):
            bj = sid + NS * k

            @pl.when(bj < nzb)
            def _():
                zero_copy(bj).wait()

        plsc.subcore_barrier()

        base = wid * nfull * CHUNK
        ones16 = jnp.ones((LANES,), jnp.float32)

        def count_dst(dst_ref):
            nlanes = dst_ref.shape[0]

            @pl.loop(0, nlanes // LANES)
            def _(j):
                idx16 = dst_ref[pl.ds(j * LANES, LANES)]
                plsc.addupdate_scatter(cnt_v, [idx16], ones16)

        def idx_copy(ci, s):
            return pltpu.make_async_copy(
                edge_hbm.at[:, pl.ds(base + ci * CHUNK, CHUNK)],
                ev.at[s], sem_i[s % 2])

        def gather_copy(b, s):
            return pltpu.make_async_copy(
                x_hbm.at[ev.at[s, 0]], rows.at[b], sem_g[b])

        def scat_copy(b, s):
            return pltpu.make_async_copy(
                rows.at[b], agg_sh.at[ev.at[s, 1]], sem_s[b])

        def scat_start(b, s):
            pltpu.async_copy(rows.at[b], agg_sh.at[ev.at[s, 1]], sem_s[b],
                             add=True)

        idx_copy(0, 0).start()
        idx_copy(1, 1).start()

        @pl.loop(0, nfull - 2, step=4)
        def _(i):
            for off in range(4):
                ci = i + off
                b = off % 2
                s = off
                pb = 1 - b
                ps = (off + 3) % 4
                idx_copy(ci, s).wait()

                @pl.when(ci >= 2)
                def _():
                    scat_copy(b, (off + 2) % 4).wait()

                gather_copy(b, s).start()

                @pl.when(ci > 0)
                def _():
                    gather_copy(pb, ps).wait()
                    scat_start(pb, ps)
                    count_dst(ev.at[ps, 1])

                idx_copy(ci + 2, (off + 2) % 4).start()

        for ci in (nfull - 2, nfull - 1):
            b, s = ci % 2, ci % 4
            pb, ps = 1 - ci % 2, (ci + 3) % 4
            idx_copy(ci, s).wait()
            scat_copy(b, (ci + 2) % 4).wait()
            gather_copy(b, s).start()
            gather_copy(pb, ps).wait()
            scat_start(pb, ps)
            count_dst(ev.at[ps, 1])
        lb, ls = (nfull - 1) % 2, (nfull - 1) % 4
        gather_copy(lb, ls).wait()
        scat_start(lb, ls)
        count_dst(ev.at[ls, 1])
        scat_copy(1 - lb, (ls + 3) % 4).wait()
        scat_copy(lb, ls).wait()

        if nextra:
            @pl.when(wid < nextra)
            def _():
                xoff = (nfull * NW + wid) * CHUNK
                pltpu.sync_copy(edge_hbm.at[:, pl.ds(xoff, CHUNK)], ev.at[0])
                pltpu.async_copy(x_hbm.at[ev.at[0, 0]], rows.at[0], sg0).wait()
                pltpu.sync_copy(rows.at[0], agg_sh.at[ev.at[0, 1]], add=True)
                count_dst(ev.at[0, 1])

        plsc.subcore_barrier()

        nwb = n // WROWS

        def wout_copy():
            rsl = pl.ds(sid * WROWS, WROWS)
            return pltpu.make_async_copy(
                agg_sh.at[rsl], agg_hbm.at[cid].at[rsl], sg0)

        @pl.when(sid < nwb)
        def _():
            wout_copy().start()

        pltpu.async_copy(cnt_v, cnt_hbm.at[pl.ds(wid * n, n)], sg1)
        pltpu.make_async_copy(cnt_v, cnt_hbm.at[pl.ds(wid * n, n)],
                              sg1).wait()

        @pl.when(sid < nwb)
        def _():
            wout_copy().wait()

    return sc_kernel(edge_index, x)


def _tc_head(agg2, cnt2, x, W_l, b_l, W_r, W_score, b_score, rs, alpha):
    n, d = x.shape

    def body(agg_ref, cnt_ref, x_ref, wl_ref, bl_ref, wr_ref, ws_ref,
             bs_ref, rs_ref, alpha_ref, out_ref):
        agg = agg_ref[0] + agg_ref[1]
        cnt = jnp.sum(cnt_ref[...], axis=0).reshape(n, 1)
        mean = agg / jnp.maximum(cnt, 1.0)
        h = lax.dot_general(mean, wl_ref[...], (((1,), (1,)), ((), ())),
                            preferred_element_type=jnp.float32)
        h = h + lax.dot_general(x_ref[...], wr_ref[...],
                                (((1,), (1,)), ((), ())),
                                preferred_element_type=jnp.float32)
        h = jnp.maximum(h + bl_ref[...], 0.0)
        g = lax.dot_general(h, ws_ref[...], (((1,), (0,)), ((), ())),
                            preferred_element_type=jnp.float32)
        g = g + bs_ref[0, 0]
        a = jax.nn.sigmoid(alpha_ref[0, 0])
        out_ref[...] = a * rs_ref[...] + (1.0 - a) * g

    return pl.pallas_call(
        body,
        out_shape=jax.ShapeDtypeStruct((n, 1), jnp.float32),
    )(agg2, cnt2.reshape(NW, n), x, W_l, b_l.reshape(1, -1), W_r,
      W_score.reshape(-1, 1), b_score.reshape(1, 1), rs.reshape(n, 1),
      alpha.reshape(1, 1))


def kernel(x, edge_index, reranker_scores, W_l, b_l, W_r, W_score, b_score,
           alpha):
    n, d = x.shape
    e = edge_index.shape[1]
    agg2, cnt2 = _sc_aggregate(edge_index, x, n, e, d)
    out = _tc_head(agg2, cnt2, x, W_l, b_l, W_r, W_score, b_score,
                   reranker_scores, alpha)
    return out[:, 0]
